# Initial kernel scaffold; baseline (speedup 1.0000x reference)
import jax
import jax.numpy as jnp
from jax.experimental import pallas as pl


def kernel(x, edge_index, edge_attr, W1, b1, W2, b2):
    raise NotImplementedError("write your pallas kernel here")


# speedup vs baseline: 11.1780x; 11.1780x over previous
import jax
import jax.numpy as jnp
from jax import lax
from jax.experimental import pallas as pl
from jax.experimental.pallas import tpu as pltpu
from jax.experimental.pallas import tpu_sc as plsc

N = 10000
IN_SIZE = 128
HID_SIZE = 128
OUT_SIZE = 64
HALF = 64

NC = 2
NS = 16
NW = NC * NS
CH = 128
L = 16

_E_REAL = 320000
E_PAD = ((_E_REAL + N + NW * CH - 1) // (NW * CH)) * (NW * CH)
C_TILE = E_PAD // (NW * CH)
NP = ((N + CH - 1) // CH) * CH
DEG_ROWS = NP // CH
ROWS_TILE = NP // NS
DUMP = 80
DUMP_CHUNKS = (80, 80, 80, 80, 80, 80, 80, 72)

_MAGIC = 0x5F3759DF


def _fill_zero_2d(ref, rows, cols):
    z = jnp.zeros((L,), jnp.float32)

    def body(k, _):
        r = k // (cols // L)
        j = k % (cols // L)
        ref[r, pl.ds(j * L, L)] = z
        return 0

    lax.fori_loop(0, rows * (cols // L), body, 0)


def _zero_acc(acc_sh, stage_v, row0):
    r = row0
    for nchunk in DUMP_CHUNKS:
        pltpu.sync_copy(stage_v.at[pl.ds(0, nchunk)],
                        acc_sh.at[pl.ds(r, nchunk)])
        r += nchunk


def _scatter_pass(h_half, src_v, dst_v, norm_v, rows_v, acc_sh, sem, width):
    def chunk_body(c, _):
        pltpu.async_copy(h_half.at[src_v.at[c]], rows_v, sem).wait()

        def scale_body(j, _2):
            nvv = plsc.load_gather(
                norm_v, [jnp.full((L,), c, jnp.int32),
                         jnp.full((L,), j, jnp.int32)])
            for f in range(width // L):
                sl = pl.ds(f * L, L)
                rows_v[j, sl] = rows_v[j, sl] * nvv
            return 0

        lax.fori_loop(0, CH, scale_body, 0)
        pltpu.sync_copy(rows_v, acc_sh.at[dst_v.at[c]], add=True)
        return 0

    lax.fori_loop(0, C_TILE, chunk_body, 0)


def _dump_acc(acc_sh, stage_v, out_hbm, row0):
    r = row0
    for nchunk in DUMP_CHUNKS:
        pltpu.sync_copy(acc_sh.at[pl.ds(r, nchunk)],
                        stage_v.at[pl.ds(0, nchunk)])
        pltpu.sync_copy(stage_v.at[pl.ds(0, nchunk)],
                        out_hbm.at[pl.ds(r, nchunk)])
        r += nchunk


def _sc_layer1_body(h1a, h1b, src_h, dst_h, ew_h, rowids_h, rowids2_h,
                    p_out, norm_out,
                    src_v, dst_v, ew_v, norm_v, deg64_v, deg1_v, dinv_v,
                    rows_v, stage_v, rowids_v, rowids2_v, acc_sh, sem):
    cid = lax.axis_index("c")
    sid = lax.axis_index("s")
    wid = sid * NC + cid
    row0 = sid * ROWS_TILE

    _fill_zero_2d(stage_v, DUMP, HALF)
    _fill_zero_2d(deg64_v, NP // HALF, HALF)
    z16 = jnp.zeros((L,), jnp.float32)

    def zdeg_body(k, _):
        deg1_v[pl.ds(k * L, L)] = z16
        return 0

    lax.fori_loop(0, NP // L, zdeg_body, 0)

    _zero_acc(acc_sh, stage_v, row0)
    pltpu.sync_copy(rowids_h, rowids_v)
    pltpu.sync_copy(rowids2_h, rowids2_v)
    plsc.subcore_barrier()

    for b in range(2):
        blk = sid * 2 + b
        pltpu.sync_copy(dst_h.at[blk], dst_v)
        pltpu.sync_copy(ew_h.at[blk], ew_v)

        def deg_body(k, _):
            c = k // (CH // L)
            j = k % (CH // L)
            d16 = dst_v[c, pl.ds(j * L, L)]
            w16 = ew_v[c, pl.ds(j * L, L)]
            plsc.addupdate_scatter(deg1_v, [d16], w16)
            return 0

        lax.fori_loop(0, C_TILE * (CH // L), deg_body, 0)

    def pack_body(k, _):
        r = k // (HALF // L)
        j = k % (HALF // L)
        deg64_v[r, pl.ds(j * L, L)] = deg1_v[pl.ds(k * L, L)]
        return 0

    lax.fori_loop(0, NP // L, pack_body, 0)
    pltpu.sync_copy(deg64_v.at[pl.ds(0, CH)],
                    acc_sh.at[rowids_v.at[0]], add=True)
    pltpu.sync_copy(deg64_v.at[pl.ds(CH, NP // HALF - CH)],
                    acc_sh.at[rowids2_v], add=True)
    plsc.subcore_barrier()

    pltpu.sync_copy(acc_sh.at[pl.ds(0, NP // HALF)],
                    deg64_v.at[pl.ds(0, NP // HALF)])
    half = jnp.float32(0.5)
    three_half = jnp.float32(1.5)

    def dinv_body(k, _):
        r = k // (HALF // L)
        j = k % (HALF // L)
        d = deg64_v[r, pl.ds(j * L, L)]
        i = lax.shift_right_logical(plsc.bitcast(d, jnp.int32), 1)
        y = plsc.bitcast(jnp.int32(_MAGIC) - i, jnp.float32)
        hd = half * d
        for _ in range(3):
            y = y * (three_half - hd * y * y)
        dinv_v[pl.ds(k * L, L)] = y
        return 0

    lax.fori_loop(0, NP // L, dinv_body, 0)

    pltpu.sync_copy(src_h.at[wid], src_v)
    pltpu.sync_copy(dst_h.at[wid], dst_v)
    pltpu.sync_copy(ew_h.at[wid], ew_v)

    def norm_body(k, _):
        c = k // (CH // L)
        j = k % (CH // L)
        s16 = src_v[c, pl.ds(j * L, L)]
        d16 = dst_v[c, pl.ds(j * L, L)]
        w16 = ew_v[c, pl.ds(j * L, L)]
        a = plsc.load_gather(dinv_v, [s16])
        b2 = plsc.load_gather(dinv_v, [d16])
        norm_v[c, pl.ds(j * L, L)] = a * w16 * b2
        return 0

    lax.fori_loop(0, C_TILE * (CH // L), norm_body, 0)
    pltpu.sync_copy(norm_v, norm_out.at[wid])

    _zero_acc(acc_sh, stage_v, row0)
    plsc.subcore_barrier()
    for hf, h_half in enumerate((h1a, h1b)):
        _scatter_pass(h_half, src_v, dst_v, norm_v, rows_v, acc_sh, sem, HALF)
        plsc.subcore_barrier()
        _dump_acc(acc_sh, stage_v, p_out.at[hf, cid], row0)
        plsc.subcore_barrier()
        if hf == 0:
            _fill_zero_2d(stage_v, DUMP, HALF)
            _zero_acc(acc_sh, stage_v, row0)
            plsc.subcore_barrier()


def _sc_layer2_body(h2, src_h, dst_h, norm_h,
                    q_out,
                    src_v, dst_v, norm_v, rows_v, stage_v, acc_sh, sem):
    cid = lax.axis_index("c")
    sid = lax.axis_index("s")
    wid = sid * NC + cid
    row0 = sid * ROWS_TILE

    _fill_zero_2d(stage_v, DUMP, OUT_SIZE)
    _zero_acc(acc_sh, stage_v, row0)
    plsc.subcore_barrier()

    pltpu.sync_copy(src_h.at[wid], src_v)
    pltpu.sync_copy(dst_h.at[wid], dst_v)
    pltpu.sync_copy(norm_h.at[wid], norm_v)

    _scatter_pass(h2, src_v, dst_v, norm_v, rows_v, acc_sh, sem, OUT_SIZE)
    plsc.subcore_barrier()
    _dump_acc(acc_sh, stage_v, q_out.at[cid], row0)


def _make_sc_layer1():
    mesh = plsc.VectorSubcoreMesh(core_axis_name="c", subcore_axis_name="s")
    return pl.kernel(
        _sc_layer1_body,
        out_type=[
            jax.ShapeDtypeStruct((2, NC, NP, HALF), jnp.float32),
            jax.ShapeDtypeStruct((NW, C_TILE, CH), jnp.float32),
        ],
        mesh=mesh,
        scratch_types=[
            pltpu.VMEM((C_TILE, CH), jnp.int32),
            pltpu.VMEM((C_TILE, CH), jnp.int32),
            pltpu.VMEM((C_TILE, CH), jnp.float32),
            pltpu.VMEM((C_TILE, CH), jnp.float32),
            pltpu.VMEM((NP // HALF, HALF), jnp.float32),
            pltpu.VMEM((NP,), jnp.float32),
            pltpu.VMEM((NP,), jnp.float32),
            pltpu.VMEM((CH, HALF), jnp.float32),
            pltpu.VMEM((DUMP, HALF), jnp.float32),
            pltpu.VMEM((2, CH), jnp.int32),
            pltpu.VMEM((NP // HALF - CH,), jnp.int32),
            pltpu.VMEM_SHARED((NP, HALF), jnp.float32),
            pltpu.SemaphoreType.DMA,
        ],
        compiler_params=pltpu.CompilerParams(needs_layout_passes=False, use_tc_tiling_on_sc=False),
        name="gcn_sc_layer1",
    )


def _make_sc_layer2():
    mesh = plsc.VectorSubcoreMesh(core_axis_name="c", subcore_axis_name="s")
    return pl.kernel(
        _sc_layer2_body,
        out_type=[jax.ShapeDtypeStruct((NC, NP, OUT_SIZE), jnp.float32)],
        mesh=mesh,
        scratch_types=[
            pltpu.VMEM((C_TILE, CH), jnp.int32),
            pltpu.VMEM((C_TILE, CH), jnp.int32),
            pltpu.VMEM((C_TILE, CH), jnp.float32),
            pltpu.VMEM((CH, OUT_SIZE), jnp.float32),
            pltpu.VMEM((DUMP, OUT_SIZE), jnp.float32),
            pltpu.VMEM_SHARED((NP, OUT_SIZE), jnp.float32),
            pltpu.SemaphoreType.DMA,
        ],
        compiler_params=pltpu.CompilerParams(needs_layout_passes=False, use_tc_tiling_on_sc=False),
        name="gcn_sc_layer2",
    )


_BLK = 1000


def _mm_body(x_ref, w_ref, o_ref):
    o_ref[...] = jnp.dot(x_ref[...], w_ref[...],
                         preferred_element_type=jnp.float32)


def _tc_matmul(x, w):
    m, k = x.shape
    n = w.shape[1]
    return pl.pallas_call(
        _mm_body,
        grid=(m // _BLK,),
        in_specs=[
            pl.BlockSpec((_BLK, k), lambda i: (i, 0)),
            pl.BlockSpec((k, n), lambda i: (0, 0)),
        ],
        out_specs=pl.BlockSpec((_BLK, n), lambda i: (i, 0)),
        out_shape=jax.ShapeDtypeStruct((m, n), jnp.float32),
    )(x, w)


def _mid_body(pa0_ref, pa1_ref, pb0_ref, pb1_ref, b1_ref, w2_ref,
              xe_ref, h2_ref):
    xe = jnp.concatenate(
        [pa0_ref[...] + pa1_ref[...], pb0_ref[...] + pb1_ref[...]],
        axis=1) + b1_ref[...]
    xe_ref[...] = xe
    h2_ref[...] = jnp.dot(jnp.maximum(xe, 0.0), w2_ref[...],
                          preferred_element_type=jnp.float32)


def _tc_mid(pa0, pa1, pb0, pb1, b1, w2):
    return pl.pallas_call(
        _mid_body,
        grid=(N // _BLK,),
        in_specs=[
            pl.BlockSpec((_BLK, HALF), lambda i: (i, 0)),
            pl.BlockSpec((_BLK, HALF), lambda i: (i, 0)),
            pl.BlockSpec((_BLK, HALF), lambda i: (i, 0)),
            pl.BlockSpec((_BLK, HALF), lambda i: (i, 0)),
            pl.BlockSpec((1, HID_SIZE), lambda i: (0, 0)),
            pl.BlockSpec((HID_SIZE, OUT_SIZE), lambda i: (0, 0)),
        ],
        out_specs=[
            pl.BlockSpec((_BLK, HID_SIZE), lambda i: (i, 0)),
            pl.BlockSpec((_BLK, OUT_SIZE), lambda i: (i, 0)),
        ],
        out_shape=[
            jax.ShapeDtypeStruct((N, HID_SIZE), jnp.float32),
            jax.ShapeDtypeStruct((N, OUT_SIZE), jnp.float32),
        ],
    )(pa0, pa1, pb0, pb1, b1, w2)


def _out_body(q0_ref, q1_ref, b2_ref, o_ref):
    o_ref[...] = q0_ref[...] + q1_ref[...] + b2_ref[...]


def _tc_out(q0, q1, b2):
    return pl.pallas_call(
        _out_body,
        grid=(N // _BLK,),
        in_specs=[
            pl.BlockSpec((_BLK, OUT_SIZE), lambda i: (i, 0)),
            pl.BlockSpec((_BLK, OUT_SIZE), lambda i: (i, 0)),
            pl.BlockSpec((1, OUT_SIZE), lambda i: (0, 0)),
        ],
        out_specs=pl.BlockSpec((_BLK, OUT_SIZE), lambda i: (i, 0)),
        out_shape=jax.ShapeDtypeStruct((N, OUT_SIZE), jnp.float32),
    )(q0, q1, b2)


@jax.jit
def kernel(x, edge_index, edge_attr, W1, b1, W2, b2):
    e = edge_index.shape[1]
    n = x.shape[0]
    pad = E_PAD - e - n
    loop = jnp.arange(n, dtype=edge_index.dtype)
    zpad_i = jnp.zeros((pad,), edge_index.dtype)
    src_f = jnp.concatenate([edge_index[0], loop, zpad_i]).reshape(NW, C_TILE, CH)
    dst_f = jnp.concatenate([edge_index[1], loop, zpad_i]).reshape(NW, C_TILE, CH)
    ew_f = jnp.concatenate([
        edge_attr, jnp.ones((n,), edge_attr.dtype),
        jnp.zeros((pad,), edge_attr.dtype)]).reshape(NW, C_TILE, CH)
    row_ids = jnp.arange(2 * CH, dtype=jnp.int32).reshape(2, CH)
    row_ids2 = jnp.arange(CH, NP // HALF, dtype=jnp.int32)

    h1 = _tc_matmul(x, W1)
    h1a = h1[:, :HALF]
    h1b = h1[:, HALF:]
    p, norm = _make_sc_layer1()(h1a, h1b, src_f, dst_f, ew_f, row_ids, row_ids2)
    x_emb, h2 = _tc_mid(p[0, 0, :N], p[0, 1, :N], p[1, 0, :N], p[1, 1, :N],
                        b1.reshape(1, HID_SIZE), W2)
    (q,) = _make_sc_layer2()(h2, src_f, dst_f, norm)
    out = _tc_out(q[0, :N], q[1, :N], b2.reshape(1, OUT_SIZE))
    return (out, x_emb)
